# baseline (speedup 1.0000x reference)
import jax
import jax.numpy as jnp
from jax import lax
from jax.experimental import pallas as pl
from jax.experimental.pallas import tpu as pltpu
from jax.experimental.pallas import tpu_sc as plsc

VOCAB = 1000000
SEQ = 200
EMBED = 64
BATCH = 4096
NC, NS = 2, 16
NW = NC * NS
BBLK = BATCH // NW
CT = EMBED // 8
BT = BATCH // 128
LANES = 16


def _body(tok_hbm, table_hbm, posr_hbm, out_hbm,
          tok_v, tok_t, rows0, rows1, outt0, outt1, poss0, poss1,
          sg0, sg1, ss0, ss1, sp0, sp1):
    w = lax.axis_index("s") * NC + lax.axis_index("c")
    rows = (rows0, rows1)
    outts = (outt0, outt1)
    posss = (poss0, poss1)
    sgs = (sg0, sg1)
    sss = (ss0, ss1)
    sps = (sp0, sp1)

    pltpu.sync_copy(tok_hbm.at[pl.ds(w * BBLK * SEQ, BBLK * SEQ)], tok_v)

    iota = lax.broadcasted_iota(jnp.int32, (LANES,), 0)
    iota_seq = iota * SEQ

    def tr_body(l, carry):
        for bl in range(BBLK // LANES):
            v = plsc.load_gather(tok_v, [iota_seq + (bl * LANES * SEQ + l)])
            tok_t[l, pl.ds(bl * LANES, LANES)] = v
        return carry

    lax.fori_loop(0, SEQ, tr_body, 0)

    def fire(l, pp):
        pltpu.async_copy(table_hbm.at[tok_t.at[l]], rows[pp], sgs[pp])
        pltpu.async_copy(posr_hbm.at[l], posss[pp], sps[pp])

    def wait_fire(l, pp):
        pltpu.make_async_copy(table_hbm.at[tok_t.at[l]], rows[pp],
                              sgs[pp]).wait()
        pltpu.make_async_copy(posr_hbm.at[l], posss[pp], sps[pp]).wait()

    def fire_store(l, pp):
        pltpu.async_copy(outts[pp], out_hbm.at[l, :, w], sss[pp])

    def wait_store(l, pp):
        pltpu.make_async_copy(outts[pp], out_hbm.at[l, :, w], sss[pp]).wait()

    def compute(l, pp):
        rv = rows[pp]
        ot = outts[pp]
        ps = posss[pp]

        def ct_body(ct, carry):
            for c8 in range(8):
                c = ct * 8 + c8
                p = ps[ct, c8, :]
                cvec = jnp.full((LANES,), c, jnp.int32)
                for bl in range(BBLK // LANES):
                    g = plsc.load_gather(rv, [bl * LANES + iota, cvec])
                    ot[ct, c8, pl.ds(bl * LANES, LANES)] = g + p
            return carry

        lax.fori_loop(0, CT, ct_body, 0)

    fire(0, 0)

    def outer(l2, carry):
        for b in range(2):
            l = 2 * l2 + b
            pp = b

            @pl.when(l >= 2)
            def _():
                wait_store(l - 2, pp)

            wait_fire(l, pp)

            @pl.when(l + 1 < SEQ)
            def _():
                fire(l + 1, 1 - pp)

            compute(l, pp)
            fire_store(l, pp)
        return carry

    lax.fori_loop(0, SEQ // 2, outer, 0)
    wait_store(SEQ - 2, 0)
    wait_store(SEQ - 1, 1)


_grid_kernel = pl.kernel(
    _body,
    out_type=jax.ShapeDtypeStruct((SEQ, CT, BT, 8, 128), jnp.float32),
    mesh=plsc.VectorSubcoreMesh(core_axis_name="c", subcore_axis_name="s"),
    scratch_types=[
        pltpu.VMEM((BBLK * SEQ,), jnp.int32),
        pltpu.VMEM((SEQ, BBLK), jnp.int32),
        pltpu.VMEM((BBLK, EMBED), jnp.float32),
        pltpu.VMEM((BBLK, EMBED), jnp.float32),
        pltpu.VMEM((CT, 8, 128), jnp.float32),
        pltpu.VMEM((CT, 8, 128), jnp.float32),
        pltpu.VMEM((CT, 8, LANES), jnp.float32),
        pltpu.VMEM((CT, 8, LANES), jnp.float32),
        pltpu.SemaphoreType.DMA,
        pltpu.SemaphoreType.DMA,
        pltpu.SemaphoreType.DMA,
        pltpu.SemaphoreType.DMA,
        pltpu.SemaphoreType.DMA,
        pltpu.SemaphoreType.DMA,
    ],
    compiler_params=pltpu.CompilerParams(use_tc_tiling_on_sc=False,
                                         needs_layout_passes=False),
)


@jax.jit
def kernel(tokens, word_table, pos_table):
    tok = tokens.astype(jnp.int32).reshape(-1)
    pos_rep = jnp.broadcast_to(
        pos_table.reshape(SEQ, CT, 8, 1), (SEQ, CT, 8, LANES))
    out5 = _grid_kernel(tok, word_table, pos_rep)
    return out5.transpose(2, 4, 0, 1, 3).reshape(BATCH, SEQ, EMBED)


# speedup vs baseline: 1.5260x; 1.5260x over previous
import jax
import jax.numpy as jnp
from jax import lax
from jax.experimental import pallas as pl
from jax.experimental.pallas import tpu as pltpu
from jax.experimental.pallas import tpu_sc as plsc

VOCAB = 1000000
SEQ = 200
EMBED = 64
BATCH = 4096
NC, NS = 2, 16
NW = NC * NS
BBLK = BATCH // NW
LANES = 16
LT = SEQ // 8


def _body(tok_hbm, table_hbm, pos_hbm, out_hbm,
          tok_v, pos_v, rows0, rows1, sg0, sg1, ss0, ss1):
    w = lax.axis_index("s") * NC + lax.axis_index("c")
    rows = (rows0, rows1)
    sgs = (sg0, sg1)
    sss = (ss0, ss1)

    pltpu.sync_copy(tok_hbm.at[:, w], tok_v)
    pltpu.sync_copy(pos_hbm, pos_v)

    def fire(l, pp):
        pltpu.async_copy(table_hbm.at[tok_v.at[l // 8, l % 8]],
                         rows[pp], sgs[pp])

    def wait_fire(l, pp):
        pltpu.make_async_copy(table_hbm.at[tok_v.at[l // 8, l % 8]],
                              rows[pp], sgs[pp]).wait()

    def fire_store(l, pp):
        pltpu.async_copy(rows[pp], out_hbm.at[l, pl.ds(w * BBLK, BBLK)],
                         sss[pp])

    def wait_store(l, pp):
        pltpu.make_async_copy(rows[pp],
                              out_hbm.at[l, pl.ds(w * BBLK, BBLK)],
                              sss[pp]).wait()

    def compute(l, pp):
        rv = rows[pp]
        ps = tuple(pos_v[l, pl.ds(c * LANES, LANES)]
                   for c in range(EMBED // LANES))

        def row_body(r, carry):
            for c in range(EMBED // LANES):
                plsc.addupdate(rv.at[r, pl.ds(c * LANES, LANES)], carry[c])
            return carry

        lax.fori_loop(0, BBLK, row_body, ps)

    fire(0, 0)

    def outer(l2, carry):
        for b in range(2):
            l = 2 * l2 + b
            pp = b

            @pl.when(l >= 2)
            def _():
                wait_store(l - 2, pp)

            wait_fire(l, pp)

            @pl.when(l + 1 < SEQ)
            def _():
                fire(l + 1, 1 - pp)

            compute(l, pp)
            fire_store(l, pp)
        return carry

    lax.fori_loop(0, SEQ // 2, outer, 0)
    wait_store(SEQ - 2, 0)
    wait_store(SEQ - 1, 1)


_grid_kernel = pl.kernel(
    _body,
    out_type=jax.ShapeDtypeStruct((SEQ, BATCH, EMBED), jnp.float32),
    mesh=plsc.VectorSubcoreMesh(core_axis_name="c", subcore_axis_name="s"),
    scratch_types=[
        pltpu.VMEM((LT, 8, BBLK), jnp.int32),
        pltpu.VMEM((SEQ, EMBED), jnp.float32),
        pltpu.VMEM((BBLK, EMBED), jnp.float32),
        pltpu.VMEM((BBLK, EMBED), jnp.float32),
        pltpu.SemaphoreType.DMA,
        pltpu.SemaphoreType.DMA,
        pltpu.SemaphoreType.DMA,
        pltpu.SemaphoreType.DMA,
    ],
    compiler_params=pltpu.CompilerParams(use_tc_tiling_on_sc=False),
)


@jax.jit
def kernel(tokens, word_table, pos_table):
    tok4 = tokens.astype(jnp.int32).T.reshape(LT, 8, NW, BBLK)
    tok4 = tok4.transpose(0, 2, 1, 3)
    out3 = _grid_kernel(tok4, word_table, pos_table)
    return out3.transpose(1, 0, 2)


# speedup vs baseline: 1.6100x; 1.0550x over previous
import jax
import jax.numpy as jnp
from jax import lax
from jax.experimental import pallas as pl
from jax.experimental.pallas import tpu as pltpu
from jax.experimental.pallas import tpu_sc as plsc

VOCAB = 1000000
SEQ = 200
EMBED = 64
BATCH = 4096
NC, NS = 2, 16
NW = NC * NS
BBLK = BATCH // NW
LANES = 16
LT = SEQ // 8
NV = EMBED // LANES


def _body(tok_hbm, table_hbm, pos_hbm, out_hbm,
          tok_v, idxb_v, pos_v, rg0, rg1, ot0, ot1,
          sg0, sg1, ss0, ss1):
    w = lax.axis_index("s") * NC + lax.axis_index("c")
    rgs = (rg0, rg1)
    ots = (ot0, ot1)
    sgs = (sg0, sg1)
    sss = (ss0, ss1)

    pltpu.sync_copy(tok_hbm.at[:, w], tok_v)
    pltpu.sync_copy(pos_hbm, pos_v)

    def cv_body(lt, carry):
        for l8 in range(8):
            for k in range(BBLK // LANES):
                sl = pl.ds(k * LANES, LANES)
                t2 = tok_v[lt, l8, sl] + tok_v[lt, l8, sl]
                tok_v[lt, l8, sl] = t2
                idxb_v[lt, l8, sl] = t2 + 1
        return carry

    lax.fori_loop(0, LT, cv_body, 0)

    def fire(l, pp):
        lt = l // 8
        l8 = l % 8
        pltpu.async_copy(table_hbm.at[tok_v.at[lt, l8]],
                         rgs[pp].at[0], sgs[pp])
        pltpu.async_copy(table_hbm.at[idxb_v.at[lt, l8]],
                         rgs[pp].at[1], sgs[pp])

    def wait_fire(l, pp):
        lt = l // 8
        l8 = l % 8
        pltpu.make_async_copy(table_hbm.at[tok_v.at[lt, l8]],
                              rgs[pp].at[0], sgs[pp]).wait()
        pltpu.make_async_copy(table_hbm.at[idxb_v.at[lt, l8]],
                              rgs[pp].at[1], sgs[pp]).wait()

    def fire_store(l, pp):
        pltpu.async_copy(ots[pp], out_hbm.at[l, pl.ds(w * (BBLK // 2),
                                                      BBLK // 2)], sss[pp])

    def wait_store(l, pp):
        pltpu.make_async_copy(ots[pp],
                              out_hbm.at[l, pl.ds(w * (BBLK // 2),
                                                  BBLK // 2)],
                              sss[pp]).wait()

    def compute(l, pp):
        rg = rgs[pp]
        ot = ots[pp]
        ps = tuple(pos_v[l, pl.ds(c * LANES, LANES)] for c in range(NV))

        def row_body(b2, carry):
            for half in range(2):
                for c in range(2):
                    sl = pl.ds(c * LANES, LANES)
                    v0 = rg[0, 2 * b2 + half, sl] + carry[c]
                    v1 = rg[1, 2 * b2 + half, sl] + carry[2 + c]
                    ot[b2, pl.ds(half * EMBED + c * LANES, LANES)] = v0
                    ot[b2, pl.ds(half * EMBED + 32 + c * LANES, LANES)] = v1
            return carry

        lax.fori_loop(0, BBLK // 2, row_body, ps)

    fire(0, 0)

    def outer(l2, carry):
        for b in range(2):
            l = 2 * l2 + b
            pp = b

            @pl.when(l >= 2)
            def _():
                wait_store(l - 2, pp)

            wait_fire(l, pp)

            @pl.when(l + 1 < SEQ)
            def _():
                fire(l + 1, 1 - pp)

            compute(l, pp)
            fire_store(l, pp)
        return carry

    lax.fori_loop(0, SEQ // 2, outer, 0)
    wait_store(SEQ - 2, 0)
    wait_store(SEQ - 1, 1)


_grid_kernel = pl.kernel(
    _body,
    out_type=jax.ShapeDtypeStruct((SEQ, BATCH // 2, 2 * EMBED), jnp.float32),
    mesh=plsc.VectorSubcoreMesh(core_axis_name="c", subcore_axis_name="s"),
    scratch_types=[
        pltpu.VMEM((LT, 8, BBLK), jnp.int32),
        pltpu.VMEM((LT, 8, BBLK), jnp.int32),
        pltpu.VMEM((SEQ, EMBED), jnp.float32),
        pltpu.VMEM((2, BBLK, EMBED // 2), jnp.float32),
        pltpu.VMEM((2, BBLK, EMBED // 2), jnp.float32),
        pltpu.VMEM((BBLK // 2, 2 * EMBED), jnp.float32),
        pltpu.VMEM((BBLK // 2, 2 * EMBED), jnp.float32),
        pltpu.SemaphoreType.DMA,
        pltpu.SemaphoreType.DMA,
        pltpu.SemaphoreType.DMA,
        pltpu.SemaphoreType.DMA,
    ],
    compiler_params=pltpu.CompilerParams(use_tc_tiling_on_sc=False),
)


@jax.jit
def kernel(tokens, word_table, pos_table):
    tok4 = tokens.astype(jnp.int32).T.reshape(LT, 8, NW, BBLK)
    tok4 = tok4.transpose(0, 2, 1, 3)
    wt32 = word_table.reshape(2 * VOCAB, EMBED // 2)
    out3 = _grid_kernel(tok4, wt32, pos_table)
    return out3.reshape(SEQ, BATCH, EMBED).transpose(1, 0, 2)
